# Initial kernel scaffold; baseline (speedup 1.0000x reference)
import jax
import jax.numpy as jnp
from jax.experimental import pallas as pl


def kernel(word_table, pos_table, ner_table, deprel_table, position_table, word_rep, pos_rep, ner_rep, deprel_rep, position_rep):
    raise NotImplementedError("write your pallas kernel here")


# speedup vs baseline: 1.8497x; 1.8497x over previous
import functools

import jax
import jax.numpy as jnp
from jax import lax
from jax.experimental import pallas as pl
from jax.experimental.pallas import tpu as pltpu
from jax.experimental.pallas import tpu_sc as plsc

B, L = 1024, 200
N = B * L
WORD_D = 64
AUX_D = 32
OUT_D = WORD_D + 4 * AUX_D

NUM_CORES = 2
NUM_SUBCORES = 16
NW = NUM_CORES * NUM_SUBCORES
PER_W = N // NW
CHUNK = 128
NCHUNK = PER_W // CHUNK

_COL_OFF = (0, WORD_D, WORD_D + AUX_D, WORD_D + 2 * AUX_D, WORD_D + 3 * AUX_D)
_DIMS = (WORD_D, AUX_D, AUX_D, AUX_D, AUX_D)


def _emb_kernel(word_hbm, pos_hbm, ner_hbm, deprel_hbm, position_hbm,
                widx_hbm, pidx_hbm, nidx_hbm, didx_hbm, xidx_hbm,
                out_hbm,
                widx_v, pidx_v, nidx_v, didx_v, xidx_v,
                wrows_v, prows_v, nrows_v, drows_v, xrows_v,
                sem):
    wid = lax.axis_index("s") * NUM_CORES + lax.axis_index("c")
    base = wid * PER_W

    pltpu.sync_copy(widx_hbm.at[pl.ds(base, PER_W)], widx_v)
    pltpu.sync_copy(pidx_hbm.at[pl.ds(base, PER_W)], pidx_v)
    pltpu.sync_copy(nidx_hbm.at[pl.ds(base, PER_W)], nidx_v)
    pltpu.sync_copy(didx_hbm.at[pl.ds(base, PER_W)], didx_v)
    pltpu.sync_copy(xidx_hbm.at[pl.ds(base, PER_W)], xidx_v)

    tables = (word_hbm, pos_hbm, ner_hbm, deprel_hbm, position_hbm)
    idxs = (widx_v, pidx_v, nidx_v, didx_v, xidx_v)
    rows = (wrows_v, prows_v, nrows_v, drows_v, xrows_v)

    @pl.loop(0, NCHUNK)
    def _(i):
        t = i * CHUNK
        row0 = base + t
        copies = []
        for tab, idx, buf in zip(tables, idxs, rows):
            copies.append(
                pltpu.async_copy(tab.at[idx.at[pl.ds(t, CHUNK)]], buf, sem))
        for c in copies:
            c.wait()
        for buf, off, dim in zip(rows, _COL_OFF, _DIMS):
            pltpu.sync_copy(buf, out_hbm.at[pl.ds(row0, CHUNK), pl.ds(off, dim)])


@jax.jit
def kernel(word_table, pos_table, ner_table, deprel_table, position_table,
           word_rep, pos_rep, ner_rep, deprel_rep, position_rep):
    mesh = plsc.VectorSubcoreMesh(core_axis_name="c", subcore_axis_name="s")
    run = pl.kernel(
        _emb_kernel,
        out_type=jax.ShapeDtypeStruct((N, OUT_D), jnp.float32),
        mesh=mesh,
        compiler_params=pltpu.CompilerParams(use_tc_tiling_on_sc=False),
        scratch_types=(
            [pltpu.VMEM((PER_W,), jnp.int32) for _ in range(5)]
            + [pltpu.VMEM((CHUNK, d), jnp.float32) for d in _DIMS]
            + [pltpu.SemaphoreType.DMA]
        ),
    )
    out = run(
        word_table, pos_table, ner_table, deprel_table, position_table,
        word_rep.reshape(N).astype(jnp.int32),
        pos_rep.reshape(N).astype(jnp.int32),
        ner_rep.reshape(N).astype(jnp.int32),
        deprel_rep.reshape(N).astype(jnp.int32),
        position_rep.reshape(N).astype(jnp.int32),
    )
    return out.reshape(B, L, OUT_D)
